# baseline (speedup 1.0000x reference)
import jax
import jax.numpy as jnp
from jax import lax
from jax.experimental import pallas as pl
from jax.experimental.pallas import tpu as pltpu
from jax.experimental.pallas import tpu_sc as plsc

B, L, D = 16, 4096, 768
NSEG = 128
LANES = 16
DH = D // 2
JV = DH // LANES
CHUNK = 64
NCHUNK = L // CHUNK


def _body(x_hbm, idx_hbm, out_hbm, xb0, xb1, ib, acc, cnt, sem0, sem1):
    c = lax.axis_index("c")
    s = lax.axis_index("s")
    b = c * 8 + s % 8
    dh = s // 8

    zero16 = jnp.zeros((LANES,), jnp.float32)
    one16 = jnp.ones((LANES,), jnp.float32)

    col0 = dh * DH

    def xsrc(ch):
        return x_hbm.at[b, pl.ds(ch * CHUNK, CHUNK), pl.ds(col0, DH)]

    pltpu.make_async_copy(xsrc(0), xb0, sem0).start()

    def zero_row(r, _):
        for j in range(JV):
            acc[r, pl.ds(j * LANES, LANES)] = zero16
        cnt[r, pl.ds(0, LANES)] = zero16
        return 0
    lax.fori_loop(0, NSEG, zero_row, 0)

    pltpu.sync_copy(idx_hbm.at[b, :], ib)

    def process(xb, ch):
        def load_row(row):
            return [xb[row, pl.ds(j * LANES, LANES)] for j in range(JV)]

        @plsc.parallel_loop(0, CHUNK // LANES)
        def grp(g):
            t0 = ch * CHUNK + g * LANES
            ids = ib[pl.ds(t0, LANES)]
            vs = load_row(g * LANES)
            for l in range(LANES):
                seg = ids[l]
                nxt = load_row(g * LANES + l + 1) if l + 1 < LANES else None
                plsc.addupdate(cnt.at[seg, pl.ds(0, LANES)], one16)
                for j in range(JV):
                    plsc.addupdate(acc.at[seg, pl.ds(j * LANES, LANES)],
                                   vs[j])
                vs = nxt

    def pair_body(p, _):
        ch0 = 2 * p
        pltpu.make_async_copy(xsrc(ch0 + 1), xb1, sem1).start()
        pltpu.make_async_copy(xsrc(ch0), xb0, sem0).wait()
        process(xb0, ch0)
        @pl.when(p < NCHUNK // 2 - 1)
        def _():
            pltpu.make_async_copy(xsrc(ch0 + 2), xb0, sem0).start()
        pltpu.make_async_copy(xsrc(ch0 + 1), xb1, sem1).wait()
        process(xb1, ch0 + 1)
        return 0
    lax.fori_loop(0, NCHUNK // 2, pair_body, 0)

    def fin_row(r, _):
        rinv = 1.0 / jnp.maximum(cnt[r, pl.ds(0, LANES)], 1.0)
        for j in range(JV):
            acc[r, pl.ds(j * LANES, LANES)] = (
                acc[r, pl.ds(j * LANES, LANES)] * rinv)
        return 0
    lax.fori_loop(0, NSEG, fin_row, 0)

    pltpu.sync_copy(acc, out_hbm.at[b, :, pl.ds(col0, DH)])


def kernel(x, sentence_idx):
    mesh = plsc.VectorSubcoreMesh(core_axis_name="c", subcore_axis_name="s")
    f = pl.kernel(
        _body,
        out_type=jax.ShapeDtypeStruct((B, NSEG, D), jnp.float32),
        mesh=mesh,
        compiler_params=pltpu.CompilerParams(needs_layout_passes=False),
        scratch_types=[
            pltpu.VMEM((CHUNK, DH), jnp.float32),
            pltpu.VMEM((CHUNK, DH), jnp.float32),
            pltpu.VMEM((L,), jnp.int32),
            pltpu.VMEM((NSEG, DH), jnp.float32),
            pltpu.VMEM((NSEG, LANES), jnp.float32),
            pltpu.SemaphoreType.DMA,
            pltpu.SemaphoreType.DMA,
        ],
    )
    return f(x, sentence_idx.astype(jnp.int32))


# speedup vs baseline: 1.3905x; 1.3905x over previous
import jax
import jax.numpy as jnp
from jax import lax
from jax.experimental import pallas as pl
from jax.experimental.pallas import tpu as pltpu
from jax.experimental.pallas import tpu_sc as plsc

B, L, D = 16, 4096, 768
NSEG = 128
LANES = 16
DH = D // 2
JV = DH // LANES
CHUNK = 64
NCHUNK = L // CHUNK


def _body(x_hbm, idx_hbm, out_hbm, xb0, xb1, ib, acc, cnt, sem0, sem1):
    c = lax.axis_index("c")
    s = lax.axis_index("s")
    b = c * 8 + s % 8
    dh = s // 8

    zero16 = jnp.zeros((LANES,), jnp.float32)
    one16 = jnp.ones((LANES,), jnp.float32)

    col0 = dh * DH

    def xsrc(ch):
        return x_hbm.at[b, pl.ds(ch * CHUNK, CHUNK), pl.ds(col0, DH)]

    pltpu.make_async_copy(xsrc(0), xb0, sem0).start()

    def zero_row(r, _):
        for j in range(JV):
            acc[r, pl.ds(j * LANES, LANES)] = zero16
        cnt[r, pl.ds(0, LANES)] = zero16
        return 0
    lax.fori_loop(0, NSEG, zero_row, 0)

    pltpu.sync_copy(idx_hbm.at[b, :], ib)

    def flush(cur_seg, run_len, accs):
        @pl.when(cur_seg >= 0)
        def _():
            rl = run_len.astype(jnp.float32)
            plsc.addupdate(cnt.at[cur_seg, pl.ds(0, LANES)],
                           jnp.broadcast_to(rl, (LANES,)))
            for j in range(JV):
                plsc.addupdate(acc.at[cur_seg, pl.ds(j * LANES, LANES)],
                               accs[j])

    def token_step(carry, xb, row, seg):
        cur_seg, run_len, accs = carry
        changed = seg != cur_seg

        @pl.when(changed)
        def _():
            flush(cur_seg, run_len, accs)

        vs = [xb[row, pl.ds(j * LANES, LANES)] for j in range(JV)]
        keep = jnp.broadcast_to(~changed, (LANES,))
        accs2 = tuple(jnp.where(keep, accs[j], 0.0) + vs[j]
                      for j in range(JV))
        run_len2 = jnp.where(changed, 1, run_len + 1)
        return (seg, run_len2, accs2)

    def process(xb, ch, carry):
        def grp(g, carry):
            t0 = ch * CHUNK + g * LANES
            ids = ib[pl.ds(t0, LANES)]
            for l in range(LANES):
                carry = token_step(carry, xb, g * LANES + l, ids[l])
            return carry
        return lax.fori_loop(0, CHUNK // LANES, grp, carry)

    def pair_body(p, carry):
        ch0 = 2 * p
        pltpu.make_async_copy(xsrc(ch0 + 1), xb1, sem1).start()
        pltpu.make_async_copy(xsrc(ch0), xb0, sem0).wait()
        carry = process(xb0, ch0, carry)
        @pl.when(p < NCHUNK // 2 - 1)
        def _():
            pltpu.make_async_copy(xsrc(ch0 + 2), xb0, sem0).start()
        pltpu.make_async_copy(xsrc(ch0 + 1), xb1, sem1).wait()
        carry = process(xb1, ch0 + 1, carry)
        return carry

    zero_accs = tuple(zero16 for _ in range(JV))
    init = (jnp.int32(-1), jnp.int32(0), zero_accs)
    cur_seg, run_len, accs = lax.fori_loop(0, NCHUNK // 2, pair_body, init)
    flush(cur_seg, run_len, accs)

    def fin_row(r, _):
        rinv = 1.0 / jnp.maximum(cnt[r, pl.ds(0, LANES)], 1.0)
        for j in range(JV):
            acc[r, pl.ds(j * LANES, LANES)] = (
                acc[r, pl.ds(j * LANES, LANES)] * rinv)
        return 0
    lax.fori_loop(0, NSEG, fin_row, 0)

    pltpu.sync_copy(acc, out_hbm.at[b, :, pl.ds(col0, DH)])


def kernel(x, sentence_idx):
    mesh = plsc.VectorSubcoreMesh(core_axis_name="c", subcore_axis_name="s")
    f = pl.kernel(
        _body,
        out_type=jax.ShapeDtypeStruct((B, NSEG, D), jnp.float32),
        mesh=mesh,
        compiler_params=pltpu.CompilerParams(needs_layout_passes=False),
        scratch_types=[
            pltpu.VMEM((CHUNK, DH), jnp.float32),
            pltpu.VMEM((CHUNK, DH), jnp.float32),
            pltpu.VMEM((L,), jnp.int32),
            pltpu.VMEM((NSEG, DH), jnp.float32),
            pltpu.VMEM((NSEG, LANES), jnp.float32),
            pltpu.SemaphoreType.DMA,
            pltpu.SemaphoreType.DMA,
        ],
    )
    return f(x, sentence_idx.astype(jnp.int32))
